# Initial kernel scaffold; baseline (speedup 1.0000x reference)
import jax
import jax.numpy as jnp
from jax.experimental import pallas as pl


def kernel(bboxes_in, scores_in, dboxes_xywh):
    raise NotImplementedError("write your pallas kernel here")


# speedup vs baseline: 1.0058x; 1.0058x over previous
import jax
import jax.numpy as jnp
from jax.experimental import pallas as pl

_SCALE_XY = 0.1
_SCALE_WH = 0.2
_CRITERIA = 0.5
_MAX_OUTPUT = 100
_TOPK_CAND = 400
_N = 8732
_C = 81


def _decode_body(bb_ref, p_ref, db_ref, boxes_ref, masked_ref):
    bt = bb_ref[0]
    p = p_ref[0]
    d = db_ref[...]

    x = bt[:, 0:1] * _SCALE_XY
    y = bt[:, 1:2] * _SCALE_XY
    w = bt[:, 2:3] * _SCALE_WH
    h = bt[:, 3:4] * _SCALE_WH
    dcx, dcy, dw, dh = d[:, 0:1], d[:, 1:2], d[:, 2:3], d[:, 3:4]
    cx = x * dw + dcx
    cy = y * dh + dcy
    cw = jnp.exp(w) * dw
    ch = jnp.exp(h) * dh
    l = jnp.clip(cx - 0.5 * cw, 0.0, 1.0)
    t = jnp.clip(cy - 0.5 * ch, 0.0, 1.0)
    r = jnp.clip(cx + 0.5 * cw, 0.0, 1.0)
    b = jnp.clip(cy + 0.5 * ch, 0.0, 1.0)
    boxes_ref[0] = jnp.concatenate([l, t, r, b], axis=1)

    size_ok = ((r - l) >= (0.1 / 300)) & ((b - t) >= (0.1 / 300))
    sc = p[:, 1:]
    valid = (sc > 0.05) & size_ok
    masked_ref[0] = jnp.where(valid, sc, -1.0)


def _decode_stage(bboxes_t, probs, dboxes_xywh):
    B = bboxes_t.shape[0]
    boxes, masked = pl.pallas_call(
        _decode_body,
        grid=(B,),
        in_specs=[
            pl.BlockSpec((1, _N, 4), lambda i: (i, 0, 0)),
            pl.BlockSpec((1, _N, _C), lambda i: (i, 0, 0)),
            pl.BlockSpec((_N, 4), lambda i: (0, 0)),
        ],
        out_specs=[
            pl.BlockSpec((1, _N, 4), lambda i: (i, 0, 0)),
            pl.BlockSpec((1, _N, _C - 1), lambda i: (i, 0, 0)),
        ],
        out_shape=[
            jax.ShapeDtypeStruct((B, _N, 4), jnp.float32),
            jax.ShapeDtypeStruct((B, _N, _C - 1), jnp.float32),
        ],
    )(bboxes_t, probs, dboxes_xywh)
    return boxes, masked


def _iou_mat(boxes):
    area = jnp.clip(boxes[:, 2] - boxes[:, 0], 0.0) * jnp.clip(
        boxes[:, 3] - boxes[:, 1], 0.0)
    lt = jnp.maximum(boxes[:, None, :2], boxes[None, :, :2])
    rb = jnp.minimum(boxes[:, None, 2:], boxes[None, :, 2:])
    whi = jnp.clip(rb - lt, 0.0)
    inter = whi[..., 0] * whi[..., 1]
    union = area[:, None] + area[None, :] - inter
    return inter / jnp.maximum(union, 1e-9)


def _select_single(boxes, masked_flat):
    cand_sc, idx = jax.lax.top_k(masked_flat, _TOPK_CAND)
    node = idx // (_C - 1)
    label = idx % (_C - 1) + 1
    cand_box = boxes[node]
    cand_valid = cand_sc > 0.0
    max_coord = jnp.max(jnp.where(cand_valid[:, None], cand_box, 0.0))
    offs = label.astype(boxes.dtype) * (max_coord + 1.0)
    nms_box = cand_box + offs[:, None]
    iou = _iou_mat(nms_box)
    ar = jnp.arange(_TOPK_CAND)

    def body(i, keep):
        sup = (iou[i] > _CRITERIA) & keep[i] & (ar > i)
        return keep & (~sup)

    keep = jax.lax.fori_loop(0, _TOPK_CAND, body, cand_valid)
    final = jnp.where(keep, cand_sc, -1.0)
    out_sc, oidx = jax.lax.top_k(final, _MAX_OUTPUT)
    kept = out_sc > 0.0
    boxes_out = jnp.where(kept[:, None], cand_box[oidx], 0.0)
    labels_out = jnp.where(kept, label[oidx], 0)
    scores_out = jnp.where(kept, out_sc, 0.0)
    return boxes_out, labels_out, scores_out


def kernel(bboxes_in, scores_in, dboxes_xywh):
    B = bboxes_in.shape[0]
    bboxes_t = jnp.transpose(bboxes_in, (0, 2, 1))
    s = jnp.transpose(scores_in, (0, 2, 1))
    probs = jax.nn.softmax(s, axis=-1)
    boxes, masked = _decode_stage(bboxes_t, probs, dboxes_xywh)
    masked_flat = masked.reshape(B, -1)
    return jax.vmap(_select_single)(boxes, masked_flat)


# speedup vs baseline: 1.0357x; 1.0297x over previous
import functools

import jax
import jax.numpy as jnp
from jax.experimental import pallas as pl
from jax.experimental.pallas import tpu as pltpu

_SCALE_XY = 0.1
_SCALE_WH = 0.2
_CRITERIA = 0.5
_MAX_OUTPUT = 100
_TOPK_CAND = 400
_N = 8732
_C = 81


def _decode_body(bb_ref, p_ref, db_ref, boxes_ref, masked_ref):
    bt = bb_ref[0]
    p = p_ref[0]
    d = db_ref[...]

    x = bt[:, 0:1] * _SCALE_XY
    y = bt[:, 1:2] * _SCALE_XY
    w = bt[:, 2:3] * _SCALE_WH
    h = bt[:, 3:4] * _SCALE_WH
    dcx, dcy, dw, dh = d[:, 0:1], d[:, 1:2], d[:, 2:3], d[:, 3:4]
    cx = x * dw + dcx
    cy = y * dh + dcy
    cw = jnp.exp(w) * dw
    ch = jnp.exp(h) * dh
    l = jnp.clip(cx - 0.5 * cw, 0.0, 1.0)
    t = jnp.clip(cy - 0.5 * ch, 0.0, 1.0)
    r = jnp.clip(cx + 0.5 * cw, 0.0, 1.0)
    b = jnp.clip(cy + 0.5 * ch, 0.0, 1.0)
    boxes_ref[0] = jnp.concatenate([l, t, r, b], axis=1)

    size_ok = ((r - l) >= (0.1 / 300)) & ((b - t) >= (0.1 / 300))
    sc = p[:, 1:]
    valid = (sc > 0.05) & size_ok
    masked_ref[0] = jnp.where(valid, sc, -1.0)


def _decode_stage(bboxes_t, probs, dboxes_xywh):
    B = bboxes_t.shape[0]
    boxes, masked = pl.pallas_call(
        _decode_body,
        grid=(B,),
        in_specs=[
            pl.BlockSpec((1, _N, 4), lambda i: (i, 0, 0)),
            pl.BlockSpec((1, _N, _C), lambda i: (i, 0, 0)),
            pl.BlockSpec((_N, 4), lambda i: (0, 0)),
        ],
        out_specs=[
            pl.BlockSpec((1, _N, 4), lambda i: (i, 0, 0)),
            pl.BlockSpec((1, _N, _C - 1), lambda i: (i, 0, 0)),
        ],
        out_shape=[
            jax.ShapeDtypeStruct((B, _N, 4), jnp.float32),
            jax.ShapeDtypeStruct((B, _N, _C - 1), jnp.float32),
        ],
    )(bboxes_t, probs, dboxes_xywh)
    return boxes, masked


def _nms_body(sc_ref, boxc_ref, lab_ref, osc_ref, olab_ref, oboxc_ref,
              sup_ref):
    B, K = sc_ref.shape
    sc = sc_ref[...]
    l = boxc_ref[:, 0, :]
    t = boxc_ref[:, 1, :]
    r = boxc_ref[:, 2, :]
    bb = boxc_ref[:, 3, :]
    lab = lab_ref[...]

    valid = sc > 0.0
    zero = jnp.zeros_like(l)
    mc = jnp.maximum(
        jnp.max(jnp.where(valid, l, zero), axis=1, keepdims=True),
        jnp.max(jnp.where(valid, t, zero), axis=1, keepdims=True))
    mc = jnp.maximum(mc, jnp.max(jnp.where(valid, r, zero), axis=1,
                                 keepdims=True))
    mc = jnp.maximum(mc, jnp.max(jnp.where(valid, bb, zero), axis=1,
                                 keepdims=True))
    offs = lab * (mc + 1.0)
    nl = l + offs
    nt = t + offs
    nr = r + offs
    nb = bb + offs
    area = jnp.clip(nr - nl, 0.0, None) * jnp.clip(nb - nt, 0.0, None)

    jidx = jax.lax.broadcasted_iota(jnp.int32, (1, 1, K), 2)

    def iou_blk(i0):
        liT = jnp.transpose(nl[:, i0:i0 + 8])
        tiT = jnp.transpose(nt[:, i0:i0 + 8])
        riT = jnp.transpose(nr[:, i0:i0 + 8])
        biT = jnp.transpose(nb[:, i0:i0 + 8])
        aiT = jnp.transpose(area[:, i0:i0 + 8])
        mlx = jnp.maximum(liT[:, :, None], nl[None, :, :])
        mly = jnp.maximum(tiT[:, :, None], nt[None, :, :])
        mrx = jnp.minimum(riT[:, :, None], nr[None, :, :])
        mry = jnp.minimum(biT[:, :, None], nb[None, :, :])
        wx = jnp.clip(mrx - mlx, 0.0, None)
        wy = jnp.clip(mry - mly, 0.0, None)
        inter = wx * wy
        union = aiT[:, :, None] + area[None, :, :] - inter
        iou = inter / jnp.maximum(union, 1e-9)
        ii = i0 + jax.lax.broadcasted_iota(jnp.int32, (8, 1, 1), 0)
        sup = (iou > _CRITERIA) & (jidx > ii)
        sup_ref[pl.ds(i0, 8)] = sup.astype(jnp.int8)

    for _i0 in range(0, K, 8):
        iou_blk(_i0)

    lane = jax.lax.broadcasted_iota(jnp.int32, (B, K), 1)

    def greedy(i, keep):
        row = sup_ref[i].astype(jnp.float32)
        oh = (lane == i).astype(jnp.float32)
        ki = jnp.sum(keep * oh, axis=1, keepdims=True)
        return keep * (1.0 - row * ki)

    keep = jax.lax.fori_loop(0, K, greedy, valid.astype(jnp.float32))

    rr = jax.lax.broadcasted_iota(jnp.int32, (K, K), 0)
    cc = jax.lax.broadcasted_iota(jnp.int32, (K, K), 1)
    lt_mat = (rr <= cc).astype(jnp.float32)
    rank = jnp.dot(keep, lt_mat,
                   preferred_element_type=jnp.float32)

    M = osc_ref.shape[1]
    srange = (jax.lax.broadcasted_iota(jnp.int32, (1, M, 1), 1)
              .astype(jnp.float32) + 1.0)
    e2 = keep[:, None, :] * (rank[:, None, :] == srange).astype(jnp.float32)
    osc_ref[...] = jnp.sum(e2 * sc[:, None, :], axis=2)
    olab_ref[...] = jnp.sum(e2 * lab[:, None, :], axis=2).astype(jnp.int32)
    oboxc_ref[:, 0, :] = jnp.sum(e2 * l[:, None, :], axis=2)
    oboxc_ref[:, 1, :] = jnp.sum(e2 * t[:, None, :], axis=2)
    oboxc_ref[:, 2, :] = jnp.sum(e2 * r[:, None, :], axis=2)
    oboxc_ref[:, 3, :] = jnp.sum(e2 * bb[:, None, :], axis=2)


def _nms_stage(cand_sc, cand_box_c, cand_lab_f):
    B, _, K = cand_box_c.shape
    M = _MAX_OUTPUT
    osc, olab, oboxc = pl.pallas_call(
        _nms_body,
        out_shape=[
            jax.ShapeDtypeStruct((B, M), jnp.float32),
            jax.ShapeDtypeStruct((B, M), jnp.int32),
            jax.ShapeDtypeStruct((B, 4, M), jnp.float32),
        ],
        scratch_shapes=[pltpu.VMEM((K, B, K), jnp.int8)],
    )(cand_sc, cand_box_c, cand_lab_f)
    return osc, olab, oboxc


def kernel(bboxes_in, scores_in, dboxes_xywh):
    B = bboxes_in.shape[0]
    bboxes_t = jnp.transpose(bboxes_in, (0, 2, 1))
    s = jnp.transpose(scores_in, (0, 2, 1))
    probs = jax.nn.softmax(s, axis=-1)
    boxes, masked = _decode_stage(bboxes_t, probs, dboxes_xywh)
    masked_flat = masked.reshape(B, -1)

    cand_sc, idx = jax.lax.top_k(masked_flat, _TOPK_CAND)
    node = idx // (_C - 1)
    label = idx % (_C - 1) + 1
    cand_box = jnp.take_along_axis(boxes, node[..., None], axis=1)
    cand_box_c = jnp.transpose(cand_box, (0, 2, 1))
    lab_f = label.astype(jnp.float32)

    osc, olab, oboxc = _nms_stage(cand_sc, cand_box_c, lab_f)
    boxes_out = jnp.transpose(oboxc, (0, 2, 1))
    return boxes_out, olab, osc
